# initial kernel scaffold (unmeasured)
import functools

import jax
import jax.numpy as jnp
from jax import lax
from jax.experimental import pallas as pl
from jax.experimental.pallas import tpu as pltpu

N_Z = 4


def kernel(O, Wo):
    B, S, Hs, D = O.shape
    K = Hs * D
    N = Wo.shape[1]
    Sc = S // N_Z

    Ob = O.reshape(B, S, K).astype(jnp.bfloat16)
    Oc = Ob.reshape(B, N_Z, Sc, K).transpose(1, 0, 2, 3)
    Wb = Wo.astype(jnp.bfloat16)
    pc = jnp.einsum(
        "cbsk,kn->cbsn", Oc, Wb, preferred_element_type=jnp.bfloat16
    )

    def body(pc_ref, out_ref, comm_ref, tmp_ref, send_sem, recv_sems, copy_sem):
        mx = lax.axis_index("x")
        my = lax.axis_index("y")
        mz = lax.axis_index("z")
        left = (mz - 1) % N_Z
        right = (mz + 1) % N_Z

        barrier = pltpu.get_barrier_semaphore()
        for nz in (left, right):
            pl.semaphore_signal(
                barrier, inc=1,
                device_id=(mx, my, nz),
                device_id_type=pl.DeviceIdType.MESH,
            )
        pl.semaphore_wait(barrier, 2)

        for s in range(N_Z - 1):
            c_recv = (mz - 2 - s) % N_Z
            src = pc_ref.at[(mz - 1) % N_Z] if s == 0 else out_ref
            rdma = pltpu.make_async_remote_copy(
                src_ref=src,
                dst_ref=comm_ref.at[s],
                send_sem=send_sem,
                recv_sem=recv_sems.at[s],
                device_id=(mx, my, right),
                device_id_type=pl.DeviceIdType.MESH,
            )
            rdma.start()
            cp = pltpu.make_async_copy(pc_ref.at[c_recv], tmp_ref, copy_sem)
            cp.start()
            rdma.wait()
            cp.wait()
            out_ref[...] = tmp_ref[...] + comm_ref[s]

        @functools.partial(pl.run_scoped, sem2=pltpu.SemaphoreType.REGULAR)
        def _(sem2):
            for nz in (left, right):
                pl.semaphore_signal(
                    sem2, inc=1,
                    device_id=(mx, my, nz),
                    device_id_type=pl.DeviceIdType.MESH,
                )
            pl.semaphore_wait(sem2, 2)

    out = pl.pallas_call(
        body,
        out_shape=jax.ShapeDtypeStruct((B, Sc, N), jnp.bfloat16),
        in_specs=[pl.BlockSpec(memory_space=pltpu.ANY)],
        out_specs=pl.BlockSpec(memory_space=pltpu.VMEM),
        scratch_shapes=[
            pltpu.VMEM((N_Z - 1, B, Sc, N), jnp.bfloat16),
            pltpu.VMEM((B, Sc, N), jnp.bfloat16),
            pltpu.SemaphoreType.DMA,
            pltpu.SemaphoreType.DMA((N_Z - 1,)),
            pltpu.SemaphoreType.DMA,
        ],
        compiler_params=pltpu.CompilerParams(collective_id=0),
    )(pc)
    return out


# baseline (device time: 358055 ns/iter reference)
import functools

import jax
import jax.numpy as jnp
from jax import lax
from jax.experimental import pallas as pl
from jax.experimental.pallas import tpu as pltpu

N_Z = 4


def kernel(O, Wo):
    B, S, Hs, D = O.shape
    K = Hs * D
    N = Wo.shape[1]
    Sc = S // N_Z

    Ob = O.reshape(B, S, K).astype(jnp.bfloat16)
    Oc = Ob.reshape(B, N_Z, Sc, K).transpose(1, 0, 2, 3)
    Wb = Wo.astype(jnp.bfloat16)
    pc = jnp.einsum(
        "cbsk,kn->cbsn", Oc, Wb, preferred_element_type=jnp.bfloat16
    )

    def body(pc_ref, out_ref, comm_ref, tmp_ref, send_sem, recv_sems, copy_sem):
        mx = lax.axis_index("x")
        my = lax.axis_index("y")
        mz = lax.axis_index("z")
        left = (mz - 1) % N_Z
        right = (mz + 1) % N_Z

        barrier = pltpu.get_barrier_semaphore()
        for nz in (left, right):
            pl.semaphore_signal(
                barrier, inc=1,
                device_id=(mx, my, nz),
                device_id_type=pl.DeviceIdType.MESH,
            )
        pl.semaphore_wait(barrier, 2)

        for s in range(N_Z - 1):
            c_recv = (mz - 2 - s) % N_Z
            src = pc_ref.at[(mz - 1) % N_Z] if s == 0 else out_ref
            rdma = pltpu.make_async_remote_copy(
                src_ref=src,
                dst_ref=comm_ref.at[s],
                send_sem=send_sem,
                recv_sem=recv_sems.at[s],
                device_id=(mx, my, right),
                device_id_type=pl.DeviceIdType.MESH,
            )
            rdma.start()
            cp = pltpu.make_async_copy(pc_ref.at[c_recv], tmp_ref, copy_sem)
            cp.start()
            rdma.wait()
            cp.wait()
            out_ref[...] = tmp_ref[...] + comm_ref[s]

        @functools.partial(pl.run_scoped, sem2=pltpu.SemaphoreType.REGULAR)
        def _(sem2):
            for nz in (left, right):
                pl.semaphore_signal(
                    sem2, inc=1,
                    device_id=(mx, my, nz),
                    device_id_type=pl.DeviceIdType.MESH,
                )
            pl.semaphore_wait(sem2, 2)

    out = pl.pallas_call(
        body,
        out_shape=jax.ShapeDtypeStruct((B, Sc, N), jnp.bfloat16),
        in_specs=[pl.BlockSpec(memory_space=pl.ANY)],
        out_specs=pl.BlockSpec(memory_space=pltpu.VMEM),
        scratch_shapes=[
            pltpu.VMEM((N_Z - 1, B, Sc, N), jnp.bfloat16),
            pltpu.VMEM((B, Sc, N), jnp.bfloat16),
            pltpu.SemaphoreType.DMA,
            pltpu.SemaphoreType.DMA((N_Z - 1,)),
            pltpu.SemaphoreType.DMA,
        ],
        compiler_params=pltpu.CompilerParams(collective_id=0),
    )(pc)
    return out
